# baseline (speedup 1.0000x reference)
import jax
import jax.numpy as jnp
from jax.experimental import pallas as pl
from jax.experimental.pallas import tpu as pltpu

_BUFFER_SIZE = 4194304
_BUF_LEN = 4195840
_TAIL = _BUF_LEN - _BUFFER_SIZE


def _dma_body(x_hbm, buf_hbm, out_hbm, sem_x, sem_t):
    cx = pltpu.make_async_copy(
        x_hbm, out_hbm.at[pl.ds(_TAIL, _BUFFER_SIZE)], sem_x
    )
    ct = pltpu.make_async_copy(
        buf_hbm.at[pl.ds(_BUFFER_SIZE, _TAIL)], out_hbm.at[pl.ds(0, _TAIL)], sem_t
    )
    cx.start()
    ct.start()
    cx.wait()
    ct.wait()


def kernel(x, buffer):
    out = pl.pallas_call(
        _dma_body,
        out_shape=jax.ShapeDtypeStruct((_BUF_LEN,), jnp.float32),
        in_specs=[
            pl.BlockSpec(memory_space=pltpu.MemorySpace.HBM),
            pl.BlockSpec(memory_space=pltpu.MemorySpace.HBM),
        ],
        out_specs=pl.BlockSpec(memory_space=pltpu.MemorySpace.HBM),
        scratch_shapes=[pltpu.SemaphoreType.DMA, pltpu.SemaphoreType.DMA],
    )(x.reshape(_BUFFER_SIZE), buffer.reshape(_BUF_LEN))
    return out.reshape(1, _BUF_LEN)


# speedup vs baseline: 2.9819x; 2.9819x over previous
import jax
import jax.numpy as jnp
from jax.experimental import pallas as pl
from jax.experimental.pallas import tpu as pltpu

_BUFFER_SIZE = 4194304
_BUF_LEN = 4195840
_TAIL = _BUF_LEN - _BUFFER_SIZE

_NCHUNK = 8
_CHUNK = _BUFFER_SIZE // _NCHUNK


def _dma_body(x_hbm, buf_hbm, out_hbm, vbuf, vtail, sin, sout, stin, stout):
    def in_copy(i):
        return pltpu.make_async_copy(
            x_hbm.at[pl.ds(i * _CHUNK, _CHUNK)], vbuf.at[i], sin.at[i]
        )

    def out_copy(i):
        return pltpu.make_async_copy(
            vbuf.at[i], out_hbm.at[pl.ds(_TAIL + i * _CHUNK, _CHUNK)], sout.at[i]
        )

    tail_in = pltpu.make_async_copy(
        buf_hbm.at[pl.ds(_BUFFER_SIZE, _TAIL)], vtail, stin
    )
    tail_in.start()
    for i in range(_NCHUNK):
        in_copy(i).start()
    tail_in.wait()
    tail_out = pltpu.make_async_copy(vtail, out_hbm.at[pl.ds(0, _TAIL)], stout)
    tail_out.start()
    for i in range(_NCHUNK):
        in_copy(i).wait()
        out_copy(i).start()
    for i in range(_NCHUNK):
        out_copy(i).wait()
    tail_out.wait()


def kernel(x, buffer):
    out = pl.pallas_call(
        _dma_body,
        out_shape=jax.ShapeDtypeStruct((_BUF_LEN,), jnp.float32),
        in_specs=[
            pl.BlockSpec(memory_space=pltpu.MemorySpace.HBM),
            pl.BlockSpec(memory_space=pltpu.MemorySpace.HBM),
        ],
        out_specs=pl.BlockSpec(memory_space=pltpu.MemorySpace.HBM),
        scratch_shapes=[
            pltpu.VMEM((_NCHUNK, _CHUNK), jnp.float32),
            pltpu.VMEM((_TAIL,), jnp.float32),
            pltpu.SemaphoreType.DMA((_NCHUNK,)),
            pltpu.SemaphoreType.DMA((_NCHUNK,)),
            pltpu.SemaphoreType.DMA,
            pltpu.SemaphoreType.DMA,
        ],
    )(x.reshape(_BUFFER_SIZE), buffer.reshape(_BUF_LEN))
    return out.reshape(1, _BUF_LEN)
